# Initial kernel scaffold; baseline (speedup 1.0000x reference)
import functools

import jax
import jax.numpy as jnp
from jax import lax
from jax.experimental import pallas as pl
from jax.experimental.pallas import tpu as pltpu
from jax.experimental.pallas import tpu_sc as plsc

EMBED = 64
HIDDEN = 64

NC = 2
NS = 16
NW = NC * NS

CHUNK = 128


def _sc_gather(table, idx3, tok, nchunk):
    per_w = nchunk * CHUNK
    mesh = plsc.VectorSubcoreMesh(core_axis_name="c", subcore_axis_name="s")

    @functools.partial(
        pl.kernel,
        out_type=jax.ShapeDtypeStruct((tok, EMBED), jnp.float32),
        mesh=mesh,
        scratch_types=[
            pltpu.VMEM((nchunk, CHUNK), jnp.int32),
            pltpu.VMEM((2, CHUNK, EMBED), jnp.float32),
            pltpu.SemaphoreType.DMA,
            pltpu.SemaphoreType.DMA,
        ],
    )
    def k(table_hbm, idx_hbm, out_hbm, idx_v, rows_v, gsem, ssem):
        wid = lax.axis_index("s") * NC + lax.axis_index("c")
        base = wid * per_w
        pltpu.sync_copy(idx_hbm.at[wid], idx_v)

        def body(j, _):
            pltpu.async_copy(
                table_hbm.at[idx_v.at[j]], rows_v.at[0], gsem
            ).wait()
            pltpu.sync_copy(
                rows_v.at[0], out_hbm.at[pl.ds(base + j * CHUNK, CHUNK)]
            )
            return 0

        lax.fori_loop(0, nchunk, body, 0)

    return k(table, idx3)


def _tc_project(emb, w_t, tok):
    blk = 2048
    grid = tok // blk

    def body(emb_ref, w_ref, out_ref):
        h = jnp.dot(emb_ref[...], w_ref[...], preferred_element_type=jnp.float32)
        inv_sqrt2 = 0.70710678118654752
        out_ref[...] = 0.5 * h * (1.0 + lax.erf(h * inv_sqrt2))

    return pl.pallas_call(
        body,
        grid=(grid,),
        in_specs=[
            pl.BlockSpec((blk, EMBED), lambda i: (i, 0)),
            pl.BlockSpec((EMBED, HIDDEN), lambda i: (0, 0)),
        ],
        out_specs=pl.BlockSpec((blk, HIDDEN), lambda i: (i, 0)),
        out_shape=jax.ShapeDtypeStruct((tok, HIDDEN), jnp.float32),
    )(emb, w_t)


def kernel(x, table, W):
    b, l = x.shape
    tok = b * l
    nchunk = tok // (NW * CHUNK)
    idx3 = x.reshape(-1).astype(jnp.int32).reshape(NW, nchunk, CHUNK)
    emb = _sc_gather(table, idx3, tok, nchunk)
    out = _tc_project(emb, W.T, tok)
    return out.reshape(b, l, HIDDEN)


# speedup vs baseline: 1.1978x; 1.1978x over previous
import functools

import jax
import jax.numpy as jnp
from jax import lax
from jax.experimental import pallas as pl
from jax.experimental.pallas import tpu as pltpu
from jax.experimental.pallas import tpu_sc as plsc

EMBED = 64
HIDDEN = 64

NC = 2
NS = 16
NW = NC * NS

CHUNK = 128


def _sc_gather(table, idx3, tok, nchunk):
    per_w = nchunk * CHUNK
    mesh = plsc.VectorSubcoreMesh(core_axis_name="c", subcore_axis_name="s")

    @functools.partial(
        pl.kernel,
        out_type=jax.ShapeDtypeStruct((tok, EMBED), jnp.float32),
        mesh=mesh,
        scratch_types=[
            pltpu.VMEM((nchunk, CHUNK), jnp.int32),
            pltpu.VMEM((2, CHUNK, EMBED), jnp.float32),
            pltpu.SemaphoreType.DMA,
            pltpu.SemaphoreType.DMA,
        ],
        compiler_params=pltpu.CompilerParams(use_tc_tiling_on_sc=False),
    )
    def k(table_hbm, idx_hbm, out_hbm, idx_v, rows_v, gsem, ssem):
        wid = lax.axis_index("s") * NC + lax.axis_index("c")
        base = wid * per_w
        pltpu.sync_copy(idx_hbm.at[wid], idx_v)

        def body(j, _):
            pltpu.async_copy(
                table_hbm.at[idx_v.at[j]], rows_v.at[0], gsem
            ).wait()
            pltpu.sync_copy(
                rows_v.at[0], out_hbm.at[pl.ds(base + j * CHUNK, CHUNK)]
            )
            return 0

        lax.fori_loop(0, nchunk, body, 0)

    return k(table, idx3)


def _tc_project(emb, w_t, tok):
    blk = 2048
    grid = tok // blk

    def body(emb_ref, w_ref, out_ref):
        h = jnp.dot(emb_ref[...], w_ref[...], preferred_element_type=jnp.float32)
        inv_sqrt2 = 0.70710678118654752
        out_ref[...] = 0.5 * h * (1.0 + lax.erf(h * inv_sqrt2))

    return pl.pallas_call(
        body,
        grid=(grid,),
        in_specs=[
            pl.BlockSpec((blk, EMBED), lambda i: (i, 0)),
            pl.BlockSpec((EMBED, HIDDEN), lambda i: (0, 0)),
        ],
        out_specs=pl.BlockSpec((blk, HIDDEN), lambda i: (i, 0)),
        out_shape=jax.ShapeDtypeStruct((tok, HIDDEN), jnp.float32),
    )(emb, w_t)


def kernel(x, table, W):
    b, l = x.shape
    tok = b * l
    nchunk = tok // (NW * CHUNK)
    idx3 = x.reshape(-1).astype(jnp.int32).reshape(NW, nchunk, CHUNK)
    emb = _sc_gather(table, idx3, tok, nchunk)
    out = _tc_project(emb, W.T, tok)
    return out.reshape(b, l, HIDDEN)


# speedup vs baseline: 1.6499x; 1.3774x over previous
import functools

import jax
import jax.numpy as jnp
from jax import lax
from jax.experimental import pallas as pl
from jax.experimental.pallas import tpu as pltpu
from jax.experimental.pallas import tpu_sc as plsc

EMBED = 64
HIDDEN = 64

NC = 2
NS = 16
NW = NC * NS

CHUNK = 128


def _sc_gather(table, idx3, tok, nchunk):
    per_w = nchunk * CHUNK
    mesh = plsc.VectorSubcoreMesh(core_axis_name="c", subcore_axis_name="s")

    @functools.partial(
        pl.kernel,
        out_type=jax.ShapeDtypeStruct((tok, EMBED), jnp.float32),
        mesh=mesh,
        scratch_types=[
            pltpu.VMEM((nchunk, CHUNK), jnp.int32),
            pltpu.VMEM((2, CHUNK, EMBED), jnp.float32),
            pltpu.SemaphoreType.DMA,
            pltpu.SemaphoreType.DMA,
        ],
        compiler_params=pltpu.CompilerParams(use_tc_tiling_on_sc=False),
    )
    def k(table_hbm, idx_hbm, out_hbm, idx_v, rows_v, gsem, ssem):
        wid = lax.axis_index("s") * NC + lax.axis_index("c")
        base = wid * per_w
        pltpu.sync_copy(idx_hbm.at[wid], idx_v)

        def body(j, _):
            pltpu.async_copy(
                table_hbm.at[idx_v.at[j]], rows_v.at[0], gsem
            ).wait()
            pltpu.sync_copy(
                rows_v.at[0], out_hbm.at[pl.ds(base + j * CHUNK, CHUNK)]
            )
            return 0

        lax.fori_loop(0, nchunk, body, 0)

    return k(table, idx3)


def _tc_project(emb128, w2, rows):
    blk = 2048
    grid = rows // blk

    def body(emb_ref, w_ref, out_ref):
        h = jnp.dot(emb_ref[...], w_ref[...], preferred_element_type=jnp.float32)
        inv_sqrt2 = 0.70710678118654752
        out_ref[...] = 0.5 * h * (1.0 + lax.erf(h * inv_sqrt2))

    return pl.pallas_call(
        body,
        grid=(grid,),
        in_specs=[
            pl.BlockSpec((blk, 2 * EMBED), lambda i: (i, 0)),
            pl.BlockSpec((2 * EMBED, 2 * HIDDEN), lambda i: (0, 0)),
        ],
        out_specs=pl.BlockSpec((blk, 2 * HIDDEN), lambda i: (i, 0)),
        out_shape=jax.ShapeDtypeStruct((rows, 2 * HIDDEN), jnp.float32),
    )(emb128, w2)


def kernel(x, table, W):
    b, l = x.shape
    tok = b * l
    nchunk = tok // (NW * CHUNK)
    idx3 = x.reshape(-1).astype(jnp.int32).reshape(NW, nchunk, CHUNK)
    emb = _sc_gather(table, idx3, tok, nchunk)
    emb128 = emb.reshape(tok // 2, 2 * EMBED)
    wt = W.T
    w2 = jnp.zeros((2 * EMBED, 2 * HIDDEN), jnp.float32)
    w2 = w2.at[:EMBED, :HIDDEN].set(wt).at[EMBED:, HIDDEN:].set(wt)
    out128 = _tc_project(emb128, w2, tok // 2)
    return out128.reshape(b, l, HIDDEN)


# speedup vs baseline: 1.6694x; 1.0118x over previous
import functools

import jax
import jax.numpy as jnp
from jax import lax
from jax.experimental import pallas as pl
from jax.experimental.pallas import tpu as pltpu
from jax.experimental.pallas import tpu_sc as plsc

EMBED = 64
HIDDEN = 64

NC = 2
NS = 16
NW = NC * NS

CHUNK = 128
BLKP = 512


def _sc_gather(table, idx3, tok, nchunk):
    per_w = nchunk * CHUNK
    mesh = plsc.VectorSubcoreMesh(core_axis_name="c", subcore_axis_name="s")

    @functools.partial(
        pl.kernel,
        out_type=jax.ShapeDtypeStruct((tok, EMBED), jnp.float32),
        mesh=mesh,
        scratch_types=[
            pltpu.VMEM((nchunk, CHUNK), jnp.int32),
            pltpu.VMEM((2, CHUNK, EMBED), jnp.float32),
            pltpu.SemaphoreType.DMA,
            pltpu.SemaphoreType.DMA,
        ],
        compiler_params=pltpu.CompilerParams(use_tc_tiling_on_sc=False),
    )
    def k(table_hbm, idx_hbm, out_hbm, idx_v, rows_v, gsem, ssem):
        wid = lax.axis_index("s") * NC + lax.axis_index("c")
        base = wid * per_w
        pltpu.sync_copy(idx_hbm.at[wid], idx_v)

        def body(j, _):
            pltpu.async_copy(
                table_hbm.at[idx_v.at[j]], rows_v.at[0], gsem
            ).wait()
            pltpu.sync_copy(
                rows_v.at[0], out_hbm.at[pl.ds(base + j * CHUNK, CHUNK)]
            )
            return 0

        lax.fori_loop(0, nchunk, body, 0)

    return k(table, idx3)


def _tc_project(emb128, w2d, b, l):
    nb = b // (2 * BLKP)
    rows_per_l = b // 2

    def body(emb_ref, w_ref, out_ref):
        h = lax.dot_general(
            w_ref[...],
            emb_ref[...],
            dimension_numbers=(((1,), (1,)), ((), ())),
            preferred_element_type=jnp.float32,
        )
        inv_sqrt2 = 0.70710678118654752
        g = 0.5 * h * (1.0 + lax.erf(h * inv_sqrt2))
        out_ref[:, :BLKP] = g[:HIDDEN, :]
        out_ref[:, BLKP:] = g[HIDDEN:, :]

    return pl.pallas_call(
        body,
        grid=(l, nb),
        in_specs=[
            pl.BlockSpec(
                (BLKP, 2 * EMBED),
                lambda li, i: (li * (rows_per_l // BLKP) + i, 0),
            ),
            pl.BlockSpec((2 * EMBED, 2 * EMBED), lambda li, i: (0, 0)),
        ],
        out_specs=pl.BlockSpec((HIDDEN, 2 * BLKP), lambda li, i: (li, i)),
        out_shape=jax.ShapeDtypeStruct((l * HIDDEN, b), jnp.float32),
    )(emb128, w2d)


def kernel(x, table, W):
    b, l = x.shape
    tok = b * l
    nchunk = tok // (NW * CHUNK)
    nb = b // (2 * BLKP)

    xp = (
        x.T.astype(jnp.int32)
        .reshape(l, nb, 2, BLKP)
        .transpose(0, 1, 3, 2)
        .reshape(-1)
    )
    idx3 = xp.reshape(NW, nchunk, CHUNK)

    emb = _sc_gather(table, idx3, tok, nchunk)
    emb128 = emb.reshape(tok // 2, 2 * EMBED)

    w2d = jnp.zeros((2 * EMBED, 2 * EMBED), jnp.float32)
    w2d = w2d.at[:HIDDEN, :EMBED].set(W).at[HIDDEN:, EMBED:].set(W)

    out2d = _tc_project(emb128, w2d, b, l)
    return out2d.reshape(l, HIDDEN, b).transpose(2, 0, 1)


# speedup vs baseline: 2.2364x; 1.3396x over previous
import functools

import jax
import jax.numpy as jnp
from jax import lax
from jax.experimental import pallas as pl
from jax.experimental.pallas import tpu as pltpu
from jax.experimental.pallas import tpu_sc as plsc

EMBED = 64
HIDDEN = 64

NC = 2
NS = 16
NW = NC * NS

CHUNK = 128


def _sc_gather(table, idx3, tok, nchunk):
    per_w = nchunk * CHUNK
    mesh = plsc.VectorSubcoreMesh(core_axis_name="c", subcore_axis_name="s")

    @functools.partial(
        pl.kernel,
        out_type=jax.ShapeDtypeStruct((tok, EMBED), jnp.float32),
        mesh=mesh,
        scratch_types=[
            pltpu.VMEM((nchunk, CHUNK), jnp.int32),
            pltpu.VMEM((2, CHUNK, EMBED), jnp.float32),
            pltpu.SemaphoreType.DMA,
            pltpu.SemaphoreType.DMA,
        ],
        compiler_params=pltpu.CompilerParams(use_tc_tiling_on_sc=False),
    )
    def k(table_hbm, idx_hbm, out_hbm, idx_v, rows_v, gsem, ssem):
        wid = lax.axis_index("s") * NC + lax.axis_index("c")
        base = wid * per_w
        pltpu.sync_copy(idx_hbm.at[wid], idx_v)

        def body(j, _):
            pltpu.async_copy(
                table_hbm.at[idx_v.at[j]], rows_v.at[0], gsem
            ).wait()
            pltpu.sync_copy(
                rows_v.at[0], out_hbm.at[pl.ds(base + j * CHUNK, CHUNK)]
            )
            return 0

        lax.fori_loop(0, nchunk, body, 0)

    return k(table, idx3)


def _tc_project(emb128, w2d, b, l):
    half = b // 2

    def body(emb_ref, w_ref, out_ref):
        h = lax.dot_general(
            w_ref[...],
            emb_ref[...],
            dimension_numbers=(((1,), (1,)), ((), ())),
            preferred_element_type=jnp.float32,
        )
        inv_sqrt2 = 0.70710678118654752
        g = 0.5 * h * (1.0 + lax.erf(h * inv_sqrt2))
        out_ref[:, :half] = g[:HIDDEN, :]
        out_ref[:, half:] = g[HIDDEN:, :]

    return pl.pallas_call(
        body,
        grid=(l,),
        in_specs=[
            pl.BlockSpec((half, 2 * EMBED), lambda li: (li, 0)),
            pl.BlockSpec((2 * EMBED, 2 * EMBED), lambda li: (0, 0)),
        ],
        out_specs=pl.BlockSpec((HIDDEN, b), lambda li: (li, 0)),
        out_shape=jax.ShapeDtypeStruct((l * HIDDEN, b), jnp.float32),
    )(emb128, w2d)


def kernel(x, table, W):
    b, l = x.shape
    tok = b * l
    nchunk = tok // (NW * CHUNK)

    xp = (
        x.T.astype(jnp.int32)
        .reshape(l, 2, b // 2)
        .transpose(0, 2, 1)
        .reshape(-1)
    )
    idx3 = xp.reshape(NW, nchunk, CHUNK)

    emb = _sc_gather(table, idx3, tok, nchunk)
    emb128 = emb.reshape(tok // 2, 2 * EMBED)

    w2d = jnp.zeros((2 * EMBED, 2 * EMBED), jnp.float32)
    w2d = w2d.at[:HIDDEN, :EMBED].set(W).at[HIDDEN:, EMBED:].set(W)

    out2d = _tc_project(emb128, w2d, b, l)
    return out2d.reshape(l, HIDDEN, b).transpose(2, 0, 1)


# speedup vs baseline: 2.6048x; 1.1648x over previous
import functools

import jax
import jax.numpy as jnp
from jax import lax
from jax.experimental import pallas as pl
from jax.experimental.pallas import tpu as pltpu
from jax.experimental.pallas import tpu_sc as plsc

EMBED = 64
HIDDEN = 64

NC = 2
NS = 16
NW = NC * NS

CHUNK = 128


def _sc_gather(table, idx3, tok, nchunk, b, l):
    per_w = nchunk * CHUNK
    half = b // 2
    mesh = plsc.VectorSubcoreMesh(core_axis_name="c", subcore_axis_name="s")

    @functools.partial(
        pl.kernel,
        out_type=jax.ShapeDtypeStruct((tok // 2, 2 * EMBED), jnp.float32),
        mesh=mesh,
        scratch_types=[
            pltpu.VMEM((nchunk, CHUNK), jnp.int32),
            pltpu.VMEM((2, CHUNK, EMBED), jnp.float32),
            pltpu.SemaphoreType.DMA,
            pltpu.SemaphoreType.DMA,
        ],
        compiler_params=pltpu.CompilerParams(use_tc_tiling_on_sc=False),
    )
    def k(table_hbm, idx_hbm, out_hbm, idx_v, rows_v, gsem, ssem):
        wid = lax.axis_index("s") * NC + lax.axis_index("c")
        base = wid * per_w
        pltpu.sync_copy(idx_hbm.at[wid], idx_v)

        def body(j, _):
            pltpu.async_copy(
                table_hbm.at[idx_v.at[j]], rows_v.at[0], gsem
            ).wait()
            s = base + j * CHUNK
            li = s // b
            r = s - li * b
            p = r // half
            bp = r - p * half
            pltpu.sync_copy(
                rows_v.at[0],
                out_hbm.at[
                    pl.ds(li * half + bp, CHUNK), pl.ds(p * EMBED, EMBED)
                ],
            )
            return 0

        lax.fori_loop(0, nchunk, body, 0)

    return k(table, idx3)


def _tc_project(emb128, w2d, b, l):
    half = b // 2

    def body(emb_ref, w_ref, out_ref):
        h = lax.dot_general(
            w_ref[...],
            emb_ref[...],
            dimension_numbers=(((1,), (1,)), ((), ())),
            preferred_element_type=jnp.float32,
        )
        inv_sqrt2 = 0.70710678118654752
        g = 0.5 * h * (1.0 + lax.erf(h * inv_sqrt2))
        out_ref[:, :half] = g[:HIDDEN, :]
        out_ref[:, half:] = g[HIDDEN:, :]

    return pl.pallas_call(
        body,
        grid=(l,),
        in_specs=[
            pl.BlockSpec((half, 2 * EMBED), lambda li: (li, 0)),
            pl.BlockSpec((2 * EMBED, 2 * EMBED), lambda li: (0, 0)),
        ],
        out_specs=pl.BlockSpec((HIDDEN, b), lambda li: (li, 0)),
        out_shape=jax.ShapeDtypeStruct((l * HIDDEN, b), jnp.float32),
    )(emb128, w2d)


def kernel(x, table, W):
    b, l = x.shape
    tok = b * l
    nchunk = tok // (NW * CHUNK)

    idx3 = x.T.astype(jnp.int32).reshape(NW, nchunk, CHUNK)

    emb128 = _sc_gather(table, idx3, tok, nchunk, b, l)

    w2d = jnp.zeros((2 * EMBED, 2 * EMBED), jnp.float32)
    w2d = w2d.at[:HIDDEN, :EMBED].set(W).at[HIDDEN:, EMBED:].set(W)

    out2d = _tc_project(emb128, w2d, b, l)
    return out2d.reshape(l, HIDDEN, b).transpose(2, 0, 1)


# speedup vs baseline: 2.9287x; 1.1244x over previous
import functools

import jax
import jax.numpy as jnp
from jax import lax
from jax.experimental import pallas as pl
from jax.experimental.pallas import tpu as pltpu
from jax.experimental.pallas import tpu_sc as plsc

EMBED = 64
HIDDEN = 64

NC = 2
NS = 16
NW = NC * NS

CHUNK = 128


def _sc_gather(table, idx3, tok, nchunk, b, l):
    per_w = nchunk * CHUNK
    half = b // 2
    mesh = plsc.VectorSubcoreMesh(core_axis_name="c", subcore_axis_name="s")

    @functools.partial(
        pl.kernel,
        out_type=jax.ShapeDtypeStruct((tok // 2, 2 * EMBED), jnp.float32),
        mesh=mesh,
        scratch_types=[
            pltpu.VMEM((nchunk, CHUNK), jnp.int32),
            pltpu.VMEM((2, CHUNK, EMBED), jnp.float32),
            pltpu.SemaphoreType.DMA,
            pltpu.SemaphoreType.DMA,
        ],
        compiler_params=pltpu.CompilerParams(use_tc_tiling_on_sc=False),
    )
    def k(table_hbm, idx_hbm, out_hbm, idx_v, rows_v, gsem0, gsem1):
        wid = lax.axis_index("s") * NC + lax.axis_index("c")
        base = wid * per_w
        pltpu.sync_copy(idx_hbm.at[wid], idx_v)

        gsems = (gsem0, gsem1)
        for buf in range(2):
            pltpu.async_copy(
                table_hbm.at[idx_v.at[buf]], rows_v.at[buf], gsems[buf]
            )

        def handle(j, buf):
            pltpu.make_async_copy(
                table_hbm.at[idx_v.at[j]], rows_v.at[buf], gsems[buf]
            ).wait()
            s = base + j * CHUNK
            li = s // b
            r = s - li * b
            p = r // half
            bp = r - p * half
            pltpu.sync_copy(
                rows_v.at[buf],
                out_hbm.at[
                    pl.ds(li * half + bp, CHUNK), pl.ds(p * EMBED, EMBED)
                ],
            )

            @pl.when(j + 2 < nchunk)
            def _():
                pltpu.async_copy(
                    table_hbm.at[idx_v.at[j + 2]], rows_v.at[buf], gsems[buf]
                )

        def body(jj, _):
            handle(2 * jj, 0)
            handle(2 * jj + 1, 1)
            return 0

        lax.fori_loop(0, nchunk // 2, body, 0)

    return k(table, idx3)


def _tc_project(emb128, w2d, b, l):
    half = b // 2

    def body(emb_ref, w_ref, out_ref):
        h = lax.dot_general(
            w_ref[...],
            emb_ref[...],
            dimension_numbers=(((1,), (1,)), ((), ())),
            preferred_element_type=jnp.float32,
        )
        inv_sqrt2 = 0.70710678118654752
        g = 0.5 * h * (1.0 + lax.erf(h * inv_sqrt2))
        out_ref[:, :half] = g[:HIDDEN, :]
        out_ref[:, half:] = g[HIDDEN:, :]

    return pl.pallas_call(
        body,
        grid=(l,),
        in_specs=[
            pl.BlockSpec((half, 2 * EMBED), lambda li: (li, 0)),
            pl.BlockSpec((2 * EMBED, 2 * EMBED), lambda li: (0, 0)),
        ],
        out_specs=pl.BlockSpec((HIDDEN, b), lambda li: (li, 0)),
        out_shape=jax.ShapeDtypeStruct((l * HIDDEN, b), jnp.float32),
    )(emb128, w2d)


def kernel(x, table, W):
    b, l = x.shape
    tok = b * l
    nchunk = tok // (NW * CHUNK)

    idx3 = x.T.astype(jnp.int32).reshape(NW, nchunk, CHUNK)

    emb128 = _sc_gather(table, idx3, tok, nchunk, b, l)

    w2d = jnp.zeros((2 * EMBED, 2 * EMBED), jnp.float32)
    w2d = w2d.at[:HIDDEN, :EMBED].set(W).at[HIDDEN:, EMBED:].set(W)

    out2d = _tc_project(emb128, w2d, b, l)
    return out2d.reshape(l, HIDDEN, b).transpose(2, 0, 1)
